# baseline (speedup 1.0000x reference)
import functools

import jax
import jax.numpy as jnp
from jax import lax
from jax.experimental import pallas as pl
from jax.experimental.pallas import tpu as pltpu
from jax.experimental.pallas import tpu_sc as plsc

_VP = 100096
_SPLIT = (2, 8, 16)


def _detile_body(in_ref, out_ref):
    v = in_ref.shape[2]
    for dp in range(4):
        a = in_ref[0, 2 * dp, :].astype(jnp.bfloat16)
        b = in_ref[0, 2 * dp + 1, :].astype(jnp.bfloat16)
        au = lax.bitcast_convert_type(a, jnp.uint16).astype(jnp.uint32)
        bu = lax.bitcast_convert_type(b, jnp.uint16).astype(jnp.uint32)
        packed = au | (bu << 16)
        out_ref[pl.ds(dp * _VP, v)] = lax.bitcast_convert_type(
            packed, jnp.float32)


def _detile(tables_t, f0, n_f):
    d, v = tables_t.shape[1], tables_t.shape[2]
    grid = (n_f, d // 8)
    return pl.pallas_call(
        _detile_body,
        grid=grid,
        in_specs=[pl.BlockSpec((1, 8, v), lambda f, g: (f + f0, g, 0))],
        out_specs=pl.BlockSpec((4 * _VP,), lambda f, g: (f * (d // 8) + g,)),
        out_shape=jax.ShapeDtypeStruct((n_f * (d // 2) * _VP,), jnp.float32),
    )(tables_t)


def _make_sc_colgather(f0, n_f, d, n_b, nw):
    nf2 = n_f // 2
    du = d // 2
    qb = n_b // 4
    mesh = plsc.VectorSubcoreMesh(core_axis_name="c", subcore_axis_name="s")

    @functools.partial(
        pl.kernel,
        mesh=mesh,
        compiler_params=pltpu.CompilerParams(use_tc_tiling_on_sc=False),
        out_type=jax.ShapeDtypeStruct((n_f * du, n_b), jnp.float32),
        scratch_types=[
            pltpu.VMEM_SHARED((2, 4, _VP), jnp.float32),
            pltpu.VMEM((n_b // 4,), jnp.int32),
            pltpu.VMEM((n_b // 4,), jnp.int32),
            pltpu.VMEM((n_b // 4,), jnp.float32),
            pltpu.VMEM((n_b // 4,), jnp.float32),
            pltpu.SemaphoreType.DMA,
            pltpu.SemaphoreType.DMA,
            pltpu.SemaphoreType.DMA,
            pltpu.SemaphoreType.DMA,
            pltpu.SemaphoreType.DMA,
            pltpu.SemaphoreType.DMA,
            pltpu.SemaphoreType.DMA,
            pltpu.SemaphoreType.DMA,
        ],
    )
    def gather_kernel(tbl_hbm, catt_hbm, out_hbm, plane, idx0, idx1,
                      buf0, buf1, lsem0, lsem1, isem0, isem1,
                      gsem0, gsem1, psem0, psem1):
        c = lax.axis_index("c")
        sid = lax.axis_index("s")
        dd = sid % 4
        b0 = (sid // 4) * qb
        idxs = (idx0, idx1)
        bufs = (buf0, buf1)
        lsems = (lsem0, lsem1)
        isems = (isem0, isem1)
        gsems = (gsem0, gsem1)
        psems = (psem0, psem1)
        n_qp = nf2 * 2

        def plane_src(g):
            row0 = (c * nf2 + g // 2) * du + (g % 2) * 4
            return tbl_hbm.at[pl.ds(row0, 4)]

        def plane_issue(g, slot):
            @pl.when(sid == 0)
            def _():
                pltpu.async_copy(plane_src(g), plane.at[slot], lsems[slot])

        def plane_wait(g, slot):
            @pl.when(sid == 0)
            def _():
                pltpu.make_async_copy(plane_src(g), plane.at[slot],
                                      lsems[slot]).wait()

        def idx_load(k):
            f_loc = c * nf2 + k
            return pltpu.async_copy(
                catt_hbm.at[f0 + f_loc, pl.ds(b0, qb)], idxs[k & 1],
                isems[k & 1])

        pcopies = [None, None]
        plane_issue(0, 0)
        icopy = idx_load(0)
        for g in range(n_qp):
            slot = g & 1
            k = g // 2
            if g + 1 < n_qp:
                plane_issue(g + 1, 1 - slot)
            plane_wait(g, slot)
            if g % 2 == 0:
                icopy.wait()
            plsc.subcore_barrier()
            if pcopies[slot] is not None:
                pcopies[slot].wait()
            pltpu.async_copy(
                plane.at[slot, dd].at[idxs[k & 1]], bufs[slot],
                gsems[slot]).wait()
            row = (c * nf2 + k) * du + (g % 2) * 4 + dd
            pcopies[slot] = pltpu.async_copy(
                bufs[slot], out_hbm.at[row, pl.ds(b0, qb)], psems[slot])
            if g % 2 == 1 and k + 1 < nf2:
                icopy = idx_load(k + 1)
            plsc.subcore_barrier()
        for j in range(2):
            if pcopies[j] is not None:
                pcopies[j].wait()

    return gather_kernel


def _mlp_t_body(ct_ref, w2t_ref, b2_ref, wat_ref, wbt_ref, bo_ref, b1_ref,
                *refs):
    n_pieces = (len(refs) - 1) // 3
    xc_refs = refs[:n_pieces]
    w1t_lo_refs = refs[n_pieces:2 * n_pieces]
    w1t_hi_refs = refs[2 * n_pieces:3 * n_pieces]
    o_ref = refs[-1]
    h_cat = b1_ref[...]
    for xc, wlo, whi in zip(xc_refs, w1t_lo_refs, w1t_hi_refs):
        u = lax.bitcast_convert_type(xc[...], jnp.uint32)
        lo = lax.bitcast_convert_type(u << 16, jnp.float32)
        hi = lax.bitcast_convert_type(u & jnp.uint32(0xFFFF0000), jnp.float32)
        h_cat = h_cat + jnp.dot(wlo[...], lo,
                                preferred_element_type=jnp.float32)
        h_cat = h_cat + jnp.dot(whi[...], hi,
                                preferred_element_type=jnp.float32)
    h_cont = jnp.dot(w2t_ref[...], ct_ref[...],
                     preferred_element_type=jnp.float32) + b2_ref[...]
    h_cat = jnp.maximum(h_cat, 0.0)
    h_cont = jnp.maximum(h_cont, 0.0)
    o_ref[...] = (jnp.dot(wat_ref[...], h_cont,
                          preferred_element_type=jnp.float32)
                  + jnp.dot(wbt_ref[...], h_cat,
                            preferred_element_type=jnp.float32)
                  + bo_ref[...])


def _mlp_t(xc_pieces, cont_t, W1, b1, W2, b2, Wout, bout, block_b=2048):
    n_b = cont_t.shape[1]
    c_dim = cont_t.shape[0]
    h_dim = W1.shape[1]
    out_dim = Wout.shape[1]
    w1t = W1.T
    w2t = W2.T
    wout_at = Wout[:h_dim].T
    wout_bt = Wout[h_dim:].T
    b1_2d = b1.reshape(h_dim, 1)
    b2_2d = b2.reshape(h_dim, 1)
    bout_2d = bout.reshape(out_dim, 1)
    w1t_lo_pieces = []
    w1t_hi_pieces = []
    col = 0
    for xc in xc_pieces:
        ncols = 2 * xc.shape[0]
        w1t_lo_pieces.append(w1t[:, col:col + ncols:2])
        w1t_hi_pieces.append(w1t[:, col + 1:col + ncols:2])
        col += ncols
    grid = (n_b // block_b,)

    def full(shape):
        return pl.BlockSpec(shape, lambda i: (0, 0))

    in_specs = [
        pl.BlockSpec((c_dim, block_b), lambda i: (0, i)),
        full(w2t.shape),
        full(b2_2d.shape),
        full(wout_at.shape),
        full(wout_bt.shape),
        full(bout_2d.shape),
        full(b1_2d.shape),
    ]
    for xc in xc_pieces:
        in_specs.append(pl.BlockSpec((xc.shape[0], block_b), lambda i: (0, i)))
    for w in w1t_lo_pieces + w1t_hi_pieces:
        in_specs.append(full(w.shape))

    out_t = pl.pallas_call(
        _mlp_t_body,
        grid=grid,
        in_specs=in_specs,
        out_specs=pl.BlockSpec((out_dim, block_b), lambda i: (0, i)),
        out_shape=jax.ShapeDtypeStruct((out_dim, n_b), jnp.float32),
    )(cont_t, w2t, b2_2d, wout_at, wout_bt, bout_2d, b1_2d,
      *xc_pieces, *w1t_lo_pieces, *w1t_hi_pieces)
    return out_t.T


def kernel(cat, cont, tables, W1, b1, W2, b2, Wout, bout):
    n_f, v, d = tables.shape
    n_b = cat.shape[0]
    tables_t = jnp.transpose(tables, (0, 2, 1))
    cat_t = cat.T
    cont_t = cont.T

    info = plsc.get_sparse_core_info()
    nw = info.num_cores * info.num_subcores

    xc_pieces = []
    f0 = 0
    for nf in _SPLIT:
        scratch = _detile(tables_t, f0, nf).reshape(nf * (d // 2), _VP)
        gather = _make_sc_colgather(f0, nf, d, n_b, nw)
        xc_pieces.append(gather(scratch, cat_t))
        f0 += nf

    return _mlp_t(xc_pieces, cont_t, W1, b1, W2, b2, Wout, bout)


# speedup vs baseline: 1.2309x; 1.2309x over previous
import functools

import jax
import jax.numpy as jnp
from jax import lax
from jax.experimental import pallas as pl
from jax.experimental.pallas import tpu as pltpu
from jax.experimental.pallas import tpu_sc as plsc

_VP = 100096
_SPLIT = (2, 8, 16)


def _detile_body(in_ref, out_ref):
    v = in_ref.shape[2]
    half = jnp.uint32(0x8000)
    topm = jnp.uint32(0xFFFF0000)
    for dp in range(4):
        au = lax.bitcast_convert_type(in_ref[0, 2 * dp, :], jnp.uint32)
        bu = lax.bitcast_convert_type(in_ref[0, 2 * dp + 1, :], jnp.uint32)
        packed = ((au + half) >> 16) | ((bu + half) & topm)
        out_ref[pl.ds(dp * _VP, v)] = lax.bitcast_convert_type(
            packed, jnp.float32)


def _detile(tables_t, f0, n_f):
    d, v = tables_t.shape[1], tables_t.shape[2]
    grid = (n_f, d // 8)
    return pl.pallas_call(
        _detile_body,
        grid=grid,
        in_specs=[pl.BlockSpec((1, 8, v), lambda f, g: (f + f0, g, 0))],
        out_specs=pl.BlockSpec((4 * _VP,), lambda f, g: (f * (d // 8) + g,)),
        out_shape=jax.ShapeDtypeStruct((n_f * (d // 2) * _VP,), jnp.float32),
    )(tables_t)


def _make_sc_colgather(f0, n_f, d, n_b, nw):
    nf2 = n_f // 2
    du = d // 2
    qb = n_b // 4
    mesh = plsc.VectorSubcoreMesh(core_axis_name="c", subcore_axis_name="s")

    @functools.partial(
        pl.kernel,
        mesh=mesh,
        compiler_params=pltpu.CompilerParams(use_tc_tiling_on_sc=False),
        out_type=jax.ShapeDtypeStruct((n_f * du, n_b), jnp.float32),
        scratch_types=[
            pltpu.VMEM_SHARED((2, 4, _VP), jnp.float32),
            pltpu.VMEM((n_b // 4,), jnp.int32),
            pltpu.VMEM((n_b // 4,), jnp.int32),
            pltpu.VMEM((n_b // 4,), jnp.float32),
            pltpu.VMEM((n_b // 4,), jnp.float32),
            pltpu.SemaphoreType.DMA,
            pltpu.SemaphoreType.DMA,
            pltpu.SemaphoreType.DMA,
            pltpu.SemaphoreType.DMA,
            pltpu.SemaphoreType.DMA,
            pltpu.SemaphoreType.DMA,
            pltpu.SemaphoreType.DMA,
            pltpu.SemaphoreType.DMA,
        ],
    )
    def gather_kernel(tbl_hbm, catt_hbm, out_hbm, plane, idx0, idx1,
                      buf0, buf1, lsem0, lsem1, isem0, isem1,
                      gsem0, gsem1, psem0, psem1):
        c = lax.axis_index("c")
        sid = lax.axis_index("s")
        dd = sid % 4
        b0 = (sid // 4) * qb
        idxs = (idx0, idx1)
        bufs = (buf0, buf1)
        lsems = (lsem0, lsem1)
        isems = (isem0, isem1)
        gsems = (gsem0, gsem1)
        psems = (psem0, psem1)
        n_qp = nf2 * 2

        def plane_src(g):
            row0 = (c * nf2 + g // 2) * du + (g % 2) * 4
            return tbl_hbm.at[pl.ds(row0, 4)]

        def plane_issue(g, slot):
            @pl.when(sid == 0)
            def _():
                pltpu.async_copy(plane_src(g), plane.at[slot], lsems[slot])

        def plane_wait(g, slot):
            @pl.when(sid == 0)
            def _():
                pltpu.make_async_copy(plane_src(g), plane.at[slot],
                                      lsems[slot]).wait()

        def idx_load(k):
            f_loc = c * nf2 + k
            return pltpu.async_copy(
                catt_hbm.at[f0 + f_loc, pl.ds(b0, qb)], idxs[k & 1],
                isems[k & 1])

        pcopies = [None, None]
        plane_issue(0, 0)
        icopy = idx_load(0)
        for g in range(n_qp):
            slot = g & 1
            k = g // 2
            if g + 1 < n_qp:
                plane_issue(g + 1, 1 - slot)
            plane_wait(g, slot)
            if g % 2 == 0:
                icopy.wait()
            plsc.subcore_barrier()
            if pcopies[slot] is not None:
                pcopies[slot].wait()
            pltpu.async_copy(
                plane.at[slot, dd].at[idxs[k & 1]], bufs[slot],
                gsems[slot]).wait()
            row = (c * nf2 + k) * du + (g % 2) * 4 + dd
            pcopies[slot] = pltpu.async_copy(
                bufs[slot], out_hbm.at[row, pl.ds(b0, qb)], psems[slot])
            if g % 2 == 1 and k + 1 < nf2:
                icopy = idx_load(k + 1)
            plsc.subcore_barrier()
        for j in range(2):
            if pcopies[j] is not None:
                pcopies[j].wait()

    return gather_kernel


def _mlp_t_body(ct_ref, w2t_ref, b2_ref, wat_ref, wbt_ref, bo_ref, b1_ref,
                *refs):
    n_pieces = (len(refs) - 1) // 3
    xc_refs = refs[:n_pieces]
    w1t_lo_refs = refs[n_pieces:2 * n_pieces]
    w1t_hi_refs = refs[2 * n_pieces:3 * n_pieces]
    o_ref = refs[-1]
    h_cat = b1_ref[...]
    for xc, wlo, whi in zip(xc_refs, w1t_lo_refs, w1t_hi_refs):
        u = lax.bitcast_convert_type(xc[...], jnp.uint32)
        lo = lax.bitcast_convert_type(u << 16, jnp.float32)
        hi = lax.bitcast_convert_type(u & jnp.uint32(0xFFFF0000), jnp.float32)
        h_cat = h_cat + jnp.dot(wlo[...], lo,
                                preferred_element_type=jnp.float32)
        h_cat = h_cat + jnp.dot(whi[...], hi,
                                preferred_element_type=jnp.float32)
    h_cont = jnp.dot(w2t_ref[...], ct_ref[...],
                     preferred_element_type=jnp.float32) + b2_ref[...]
    h_cat = jnp.maximum(h_cat, 0.0)
    h_cont = jnp.maximum(h_cont, 0.0)
    o_ref[...] = (jnp.dot(wat_ref[...], h_cont,
                          preferred_element_type=jnp.float32)
                  + jnp.dot(wbt_ref[...], h_cat,
                            preferred_element_type=jnp.float32)
                  + bo_ref[...])


def _mlp_t(xc_pieces, cont_t, W1, b1, W2, b2, Wout, bout, block_b=2048):
    n_b = cont_t.shape[1]
    c_dim = cont_t.shape[0]
    h_dim = W1.shape[1]
    out_dim = Wout.shape[1]
    w1t = W1.T
    w2t = W2.T
    wout_at = Wout[:h_dim].T
    wout_bt = Wout[h_dim:].T
    b1_2d = b1.reshape(h_dim, 1)
    b2_2d = b2.reshape(h_dim, 1)
    bout_2d = bout.reshape(out_dim, 1)
    w1t_lo_pieces = []
    w1t_hi_pieces = []
    col = 0
    for xc in xc_pieces:
        ncols = 2 * xc.shape[0]
        w1t_lo_pieces.append(w1t[:, col:col + ncols:2])
        w1t_hi_pieces.append(w1t[:, col + 1:col + ncols:2])
        col += ncols
    grid = (n_b // block_b,)

    def full(shape):
        return pl.BlockSpec(shape, lambda i: (0, 0))

    in_specs = [
        pl.BlockSpec((c_dim, block_b), lambda i: (0, i)),
        full(w2t.shape),
        full(b2_2d.shape),
        full(wout_at.shape),
        full(wout_bt.shape),
        full(bout_2d.shape),
        full(b1_2d.shape),
    ]
    for xc in xc_pieces:
        in_specs.append(pl.BlockSpec((xc.shape[0], block_b), lambda i: (0, i)))
    for w in w1t_lo_pieces + w1t_hi_pieces:
        in_specs.append(full(w.shape))

    out_t = pl.pallas_call(
        _mlp_t_body,
        grid=grid,
        in_specs=in_specs,
        out_specs=pl.BlockSpec((out_dim, block_b), lambda i: (0, i)),
        out_shape=jax.ShapeDtypeStruct((out_dim, n_b), jnp.float32),
    )(cont_t, w2t, b2_2d, wout_at, wout_bt, bout_2d, b1_2d,
      *xc_pieces, *w1t_lo_pieces, *w1t_hi_pieces)
    return out_t.T


def kernel(cat, cont, tables, W1, b1, W2, b2, Wout, bout):
    n_f, v, d = tables.shape
    n_b = cat.shape[0]
    tables_t = jnp.transpose(tables, (0, 2, 1))
    cat_t = cat.T
    cont_t = cont.T

    info = plsc.get_sparse_core_info()
    nw = info.num_cores * info.num_subcores

    xc_pieces = []
    f0 = 0
    for nf in _SPLIT:
        scratch = _detile(tables_t, f0, nf).reshape(nf * (d // 2), _VP)
        gather = _make_sc_colgather(f0, nf, d, n_b, nw)
        xc_pieces.append(gather(scratch, cat_t))
        f0 += nf

    return _mlp_t(xc_pieces, cont_t, W1, b1, W2, b2, Wout, bout)


# speedup vs baseline: 1.4374x; 1.1678x over previous
import functools

import jax
import jax.numpy as jnp
from jax import lax
from jax.experimental import pallas as pl
from jax.experimental.pallas import tpu as pltpu
from jax.experimental.pallas import tpu_sc as plsc

_VP = 100096
_SPLIT = (2, 8, 16)


def _detile_body(in_ref, out_ref):
    v = in_ref.shape[2]
    topm = jnp.uint32(0xFFFF0000)
    for dp in range(4):
        au = lax.bitcast_convert_type(in_ref[0, 2 * dp, :], jnp.uint32)
        bu = lax.bitcast_convert_type(in_ref[0, 2 * dp + 1, :], jnp.uint32)
        packed = (au >> 16) | (bu & topm)
        out_ref[pl.ds(dp * _VP, v)] = lax.bitcast_convert_type(
            packed, jnp.float32)


def _detile(tables_t, f0, n_f):
    d, v = tables_t.shape[1], tables_t.shape[2]
    grid = (n_f, d // 8)
    return pl.pallas_call(
        _detile_body,
        grid=grid,
        in_specs=[pl.BlockSpec((1, 8, v), lambda f, g: (f + f0, g, 0))],
        out_specs=pl.BlockSpec((4 * _VP,), lambda f, g: (f * (d // 8) + g,)),
        out_shape=jax.ShapeDtypeStruct((n_f * (d // 2) * _VP,), jnp.float32),
    )(tables_t)


def _make_sc_colgather(f0, n_f, d, n_b, nw):
    nf2 = n_f // 2
    du = d // 2
    qb = n_b // 4
    mesh = plsc.VectorSubcoreMesh(core_axis_name="c", subcore_axis_name="s")

    @functools.partial(
        pl.kernel,
        mesh=mesh,
        compiler_params=pltpu.CompilerParams(use_tc_tiling_on_sc=False),
        out_type=jax.ShapeDtypeStruct((n_f * du, n_b), jnp.float32),
        scratch_types=[
            pltpu.VMEM_SHARED((2, 4, _VP), jnp.float32),
            pltpu.VMEM((n_b // 4,), jnp.int32),
            pltpu.VMEM((n_b // 4,), jnp.int32),
            pltpu.VMEM((n_b // 4,), jnp.float32),
            pltpu.VMEM((n_b // 4,), jnp.float32),
            pltpu.SemaphoreType.DMA,
            pltpu.SemaphoreType.DMA,
            pltpu.SemaphoreType.DMA,
            pltpu.SemaphoreType.DMA,
            pltpu.SemaphoreType.DMA,
            pltpu.SemaphoreType.DMA,
            pltpu.SemaphoreType.DMA,
            pltpu.SemaphoreType.DMA,
        ],
    )
    def gather_kernel(tbl_hbm, catt_hbm, out_hbm, plane, idx0, idx1,
                      buf0, buf1, lsem0, lsem1, isem0, isem1,
                      gsem0, gsem1, psem0, psem1):
        c = lax.axis_index("c")
        sid = lax.axis_index("s")
        dd = sid % 4
        b0 = (sid // 4) * qb
        idxs = (idx0, idx1)
        bufs = (buf0, buf1)
        lsems = (lsem0, lsem1)
        isems = (isem0, isem1)
        gsems = (gsem0, gsem1)
        psems = (psem0, psem1)
        n_qp = nf2 * 2

        def plane_src(g):
            row0 = (c * nf2 + g // 2) * du + (g % 2) * 4
            return tbl_hbm.at[pl.ds(row0, 4)]

        def plane_issue(g, slot):
            @pl.when(sid == 0)
            def _():
                pltpu.async_copy(plane_src(g), plane.at[slot], lsems[slot])

        def plane_wait(g, slot):
            @pl.when(sid == 0)
            def _():
                pltpu.make_async_copy(plane_src(g), plane.at[slot],
                                      lsems[slot]).wait()

        def idx_load(k):
            f_loc = c * nf2 + k
            return pltpu.async_copy(
                catt_hbm.at[f0 + f_loc, pl.ds(b0, qb)], idxs[k & 1],
                isems[k & 1])

        pcopies = [None, None]
        plane_issue(0, 0)
        icopy = idx_load(0)
        for g in range(n_qp):
            slot = g & 1
            k = g // 2
            if g + 1 < n_qp:
                plane_issue(g + 1, 1 - slot)
            plane_wait(g, slot)
            if g % 2 == 0:
                icopy.wait()
            plsc.subcore_barrier()
            if pcopies[slot] is not None:
                pcopies[slot].wait()
            pltpu.async_copy(
                plane.at[slot, dd].at[idxs[k & 1]], bufs[slot],
                gsems[slot]).wait()
            row = (c * nf2 + k) * du + (g % 2) * 4 + dd
            pcopies[slot] = pltpu.async_copy(
                bufs[slot], out_hbm.at[row, pl.ds(b0, qb)], psems[slot])
            if g % 2 == 1 and k + 1 < nf2:
                icopy = idx_load(k + 1)
            plsc.subcore_barrier()
        for j in range(2):
            if pcopies[j] is not None:
                pcopies[j].wait()

    return gather_kernel


def _mlp_t_body(ct_ref, w2t_ref, b2_ref, wat_ref, wbt_ref, bo_ref, b1_ref,
                *refs):
    n_pieces = (len(refs) - 1) // 3
    xc_refs = refs[:n_pieces]
    w1t_lo_refs = refs[n_pieces:2 * n_pieces]
    w1t_hi_refs = refs[2 * n_pieces:3 * n_pieces]
    o_ref = refs[-1]
    h_cat = b1_ref[...]
    for xc, wlo, whi in zip(xc_refs, w1t_lo_refs, w1t_hi_refs):
        u = lax.bitcast_convert_type(xc[...], jnp.uint32)
        lo = lax.bitcast_convert_type(u << 16, jnp.float32)
        hi = lax.bitcast_convert_type(u & jnp.uint32(0xFFFF0000), jnp.float32)
        h_cat = h_cat + jnp.dot(wlo[...], lo,
                                preferred_element_type=jnp.float32)
        h_cat = h_cat + jnp.dot(whi[...], hi,
                                preferred_element_type=jnp.float32)
    h_cont = jnp.dot(w2t_ref[...], ct_ref[...],
                     preferred_element_type=jnp.float32) + b2_ref[...]
    h_cat = jnp.maximum(h_cat, 0.0)
    h_cont = jnp.maximum(h_cont, 0.0)
    o_ref[...] = (jnp.dot(wat_ref[...], h_cont,
                          preferred_element_type=jnp.float32)
                  + jnp.dot(wbt_ref[...], h_cat,
                            preferred_element_type=jnp.float32)
                  + bo_ref[...])


def _mlp_t(xc_pieces, cont_t, W1, b1, W2, b2, Wout, bout, block_b=2048):
    n_b = cont_t.shape[1]
    c_dim = cont_t.shape[0]
    h_dim = W1.shape[1]
    out_dim = Wout.shape[1]
    w1t = W1.T
    w2t = W2.T
    wout_at = Wout[:h_dim].T
    wout_bt = Wout[h_dim:].T
    b1_2d = b1.reshape(h_dim, 1)
    b2_2d = b2.reshape(h_dim, 1)
    bout_2d = bout.reshape(out_dim, 1)
    w1t_lo_pieces = []
    w1t_hi_pieces = []
    col = 0
    for xc in xc_pieces:
        ncols = 2 * xc.shape[0]
        w1t_lo_pieces.append(w1t[:, col:col + ncols:2])
        w1t_hi_pieces.append(w1t[:, col + 1:col + ncols:2])
        col += ncols
    grid = (n_b // block_b,)

    def full(shape):
        return pl.BlockSpec(shape, lambda i: (0, 0))

    in_specs = [
        pl.BlockSpec((c_dim, block_b), lambda i: (0, i)),
        full(w2t.shape),
        full(b2_2d.shape),
        full(wout_at.shape),
        full(wout_bt.shape),
        full(bout_2d.shape),
        full(b1_2d.shape),
    ]
    for xc in xc_pieces:
        in_specs.append(pl.BlockSpec((xc.shape[0], block_b), lambda i: (0, i)))
    for w in w1t_lo_pieces + w1t_hi_pieces:
        in_specs.append(full(w.shape))

    out_t = pl.pallas_call(
        _mlp_t_body,
        grid=grid,
        in_specs=in_specs,
        out_specs=pl.BlockSpec((out_dim, block_b), lambda i: (0, i)),
        out_shape=jax.ShapeDtypeStruct((out_dim, n_b), jnp.float32),
    )(cont_t, w2t, b2_2d, wout_at, wout_bt, bout_2d, b1_2d,
      *xc_pieces, *w1t_lo_pieces, *w1t_hi_pieces)
    return out_t.T


def kernel(cat, cont, tables, W1, b1, W2, b2, Wout, bout):
    n_f, v, d = tables.shape
    n_b = cat.shape[0]
    tables_t = jnp.transpose(tables, (0, 2, 1))
    cat_t = cat.T
    cont_t = cont.T

    info = plsc.get_sparse_core_info()
    nw = info.num_cores * info.num_subcores

    xc_pieces = []
    f0 = 0
    for nf in _SPLIT:
        scratch = _detile(tables_t, f0, nf).reshape(nf * (d // 2), _VP)
        gather = _make_sc_colgather(f0, nf, d, n_b, nw)
        xc_pieces.append(gather(scratch, cat_t))
        f0 += nf

    return _mlp_t(xc_pieces, cont_t, W1, b1, W2, b2, Wout, bout)
